# Initial kernel scaffold; baseline (speedup 1.0000x reference)
import functools

import jax
import jax.numpy as jnp
from jax import lax
from jax.experimental import pallas as pl
from jax.experimental.pallas import tpu as pltpu
from jax.experimental.pallas import tpu_sc as plsc

N = 10000
NPAD = 10240
H = 16
C = 40
E = 320000
NC, NS = 2, 16
NW = NC * NS
CHUNK = 128
NCHUNK = 79
EPT = NCHUNK * CHUNK
RPT = NPAD // NS

_MESH = plsc.VectorSubcoreMesh(
    core_axis_name="c", subcore_axis_name="s", num_cores=NC, num_subcores=NS
)


@functools.partial(
    pl.kernel,
    out_type=jax.ShapeDtypeStruct((NC, NPAD), jnp.float32),
    mesh=_MESH,
    scratch_types=[
        pltpu.VMEM((NCHUNK, CHUNK), jnp.int32),
        pltpu.VMEM((CHUNK,), jnp.float32),
        pltpu.VMEM((RPT,), jnp.float32),
        pltpu.VMEM_SHARED((NPAD,), jnp.float32),
    ],
)
def _deg_kernel(dst_hbm, out_hbm, didx, ones_v, z_v, acc):
    cid = lax.axis_index("c")
    sid = lax.axis_index("s")
    wid = sid * NC + cid
    for g in range(CHUNK // 16):
        ones_v[pl.ds(g * 16, 16)] = jnp.ones((16,), jnp.float32)

    @pl.loop(0, RPT // 16)
    def _zero(i):
        z_v[pl.ds(pl.multiple_of(i * 16, 16), 16)] = jnp.zeros((16,), jnp.float32)

    pltpu.sync_copy(z_v, acc.at[pl.ds(sid * RPT, RPT)])
    pltpu.sync_copy(dst_hbm.at[wid], didx)
    plsc.subcore_barrier()

    @pl.loop(0, NCHUNK)
    def _scat(ch):
        pltpu.sync_copy(ones_v, acc.at[didx.at[ch]], add=True)

    plsc.subcore_barrier()
    pltpu.sync_copy(
        acc.at[pl.ds(sid * RPT, RPT)], out_hbm.at[cid, pl.ds(sid * RPT, RPT)]
    )


@functools.partial(
    pl.kernel,
    out_type=jax.ShapeDtypeStruct((NC, NPAD, H), jnp.float32),
    mesh=_MESH,
    scratch_types=[
        pltpu.VMEM((NCHUNK, CHUNK), jnp.int32),
        pltpu.VMEM((NCHUNK, CHUNK), jnp.int32),
        pltpu.VMEM((CHUNK, H), jnp.float32),
        pltpu.VMEM((CHUNK, H), jnp.float32),
        pltpu.VMEM((RPT, H), jnp.float32),
        pltpu.VMEM_SHARED((NPAD, H), jnp.float32),
        pltpu.SemaphoreType.DMA,
        pltpu.SemaphoreType.DMA,
    ],
)
def _agg_kernel(src_hbm, dst_hbm, tab_hbm, out_hbm,
                sidx, didx, rows0, rows1, z_v, acc, sem0, sem1):
    cid = lax.axis_index("c")
    sid = lax.axis_index("s")
    wid = sid * NC + cid

    @pl.loop(0, RPT)
    def _zero(i):
        z_v[i, :] = jnp.zeros((16,), jnp.float32)

    pltpu.sync_copy(z_v, acc.at[pl.ds(sid * RPT, RPT)])
    pltpu.sync_copy(src_hbm.at[wid], sidx)
    pltpu.sync_copy(dst_hbm.at[wid], didx)
    plsc.subcore_barrier()

    pltpu.async_copy(tab_hbm.at[sidx.at[0]], rows0, sem0)

    @pl.loop(0, (NCHUNK - 1) // 2)
    def _pair(p):
        c = pl.multiple_of(p * 2, 2)
        pltpu.make_async_copy(tab_hbm.at[sidx.at[c]], rows0, sem0).wait()
        pltpu.async_copy(tab_hbm.at[sidx.at[c + 1]], rows1, sem1)
        pltpu.sync_copy(rows0, acc.at[didx.at[c]], add=True)
        pltpu.make_async_copy(tab_hbm.at[sidx.at[c + 1]], rows1, sem1).wait()
        pltpu.async_copy(tab_hbm.at[sidx.at[c + 2]], rows0, sem0)
        pltpu.sync_copy(rows1, acc.at[didx.at[c + 1]], add=True)

    pltpu.make_async_copy(tab_hbm.at[sidx.at[NCHUNK - 1]], rows0, sem0).wait()
    pltpu.sync_copy(rows0, acc.at[didx.at[NCHUNK - 1]], add=True)

    plsc.subcore_barrier()
    pltpu.sync_copy(
        acc.at[pl.ds(sid * RPT, RPT)], out_hbm.at[cid, pl.ds(sid * RPT, RPT)]
    )


def _tc1_body(x_ref, w1_ref, degp_ref, h_ref, dis_ref, hs_ref):
    h = jnp.dot(x_ref[...], w1_ref[...], preferred_element_type=jnp.float32)
    deg = degp_ref[0, :] + degp_ref[1, :] + 1.0
    dis = lax.rsqrt(deg)[:, None]
    h_ref[...] = h
    dis_ref[...] = dis
    hs_ref[...] = h * dis


def _tc2_body(aggp_ref, h_ref, dis_ref, b1_ref, z_ref, hs_ref):
    agg = aggp_ref[0] + aggp_ref[1]
    dis = dis_ref[...]
    pre = dis * agg + (dis * dis) * h_ref[...] + b1_ref[...][None, :]
    z = jnp.maximum(pre, 0.0)
    z_ref[...] = z
    hs_ref[...] = z * dis


def _tc3_body(aggp_ref, z_ref, dis_ref, w2_ref, b2_ref, out_ref):
    agg = aggp_ref[0] + aggp_ref[1]
    dis = dis_ref[...]
    pre = dis * agg + (dis * dis) * z_ref[...]
    logits = (
        jnp.dot(pre, w2_ref[...], preferred_element_type=jnp.float32)
        + b2_ref[...][None, :]
    )
    m = jnp.max(logits, axis=1, keepdims=True)
    lse = jnp.log(jnp.sum(jnp.exp(logits - m), axis=1, keepdims=True)) + m
    out_ref[...] = logits - lse


_tc1 = pl.pallas_call(
    _tc1_body,
    out_shape=[
        jax.ShapeDtypeStruct((NPAD, H), jnp.float32),
        jax.ShapeDtypeStruct((NPAD, 1), jnp.float32),
        jax.ShapeDtypeStruct((NPAD, H), jnp.float32),
    ],
)

_tc2 = pl.pallas_call(
    _tc2_body,
    out_shape=[
        jax.ShapeDtypeStruct((NPAD, H), jnp.float32),
        jax.ShapeDtypeStruct((NPAD, H), jnp.float32),
    ],
)

_tc3 = pl.pallas_call(
    _tc3_body,
    out_shape=jax.ShapeDtypeStruct((NPAD, C), jnp.float32),
)


@jax.jit
def kernel(x, edge_index, W1, b1, W2, b2):
    src = edge_index[0].astype(jnp.int32)
    dst = edge_index[1].astype(jnp.int32)
    pad = NW * EPT - E
    srcp = jnp.concatenate([src, jnp.full((pad,), N, jnp.int32)])
    dstp = jnp.concatenate([dst, jnp.full((pad,), N, jnp.int32)])
    srcp = srcp.reshape(NW, NCHUNK, CHUNK)
    dstp = dstp.reshape(NW, NCHUNK, CHUNK)
    xp = jnp.pad(x, ((0, NPAD - N), (0, 0)))

    degp = _deg_kernel(dstp)
    h1, dis, hs1 = _tc1(xp, W1, degp)
    aggp1 = _agg_kernel(srcp, dstp, hs1)
    z1, hs2 = _tc2(aggp1, h1, dis, b1)
    aggp2 = _agg_kernel(srcp, dstp, hs2)
    outp = _tc3(aggp2, z1, dis, W2, b2)
    return outp[:N]


# speedup vs baseline: 67.9156x; 67.9156x over previous
import functools

import jax
import jax.numpy as jnp
from jax import lax
from jax.experimental import pallas as pl
from jax.experimental.pallas import tpu as pltpu
from jax.experimental.pallas import tpu_sc as plsc

N = 10000
NPAD = 10240
H = 16
C = 40
E = 320000
NC, NS = 2, 16
NW = NC * NS
CHUNK = 128
NCHUNK = 80
EPT = NCHUNK * CHUNK
NBUF = 8
DEPTH = 4
RPT = NPAD // NS

_MESH = plsc.VectorSubcoreMesh(
    core_axis_name="c", subcore_axis_name="s", num_cores=NC, num_subcores=NS
)

_SC_PARAMS = pltpu.CompilerParams(
    use_tc_tiling_on_sc=False, needs_layout_passes=False
)

_IN_BOUNDS = lax.GatherScatterMode.PROMISE_IN_BOUNDS


def _rsqrt16(x):
    xh = x * 0.5
    i = plsc.bitcast(x, jnp.int32)
    i = jnp.int32(0x5F3759DF) - (i >> 1)
    y = plsc.bitcast(i, jnp.float32)
    for _ in range(3):
        y = y * (1.5 - xh * y * y)
    return y


def _lane_bcast(v, i):
    return jnp.take_along_axis(
        v, jnp.full((16,), i, jnp.int32), axis=0, mode=_IN_BOUNDS
    )


def _dis_group(d0, d1, g):
    r0 = pl.multiple_of(g * 16, 16)
    deg = d0[pl.ds(r0, 16)] + d1[pl.ds(r0, 16)] + 1.0
    return _rsqrt16(deg)


def _edge_ring(tab_sh, acc, sidx, didx, rows, gsem, ssem):
    def gather(c, b):
        pltpu.async_copy(tab_sh.at[sidx.at[c]], rows[b], gsem[b])

    def wait_gather(c, b):
        pltpu.make_async_copy(tab_sh.at[sidx.at[c]], rows[b], gsem[b]).wait()

    def scatter(c, b):
        pltpu.async_copy(rows[b], acc.at[didx.at[c]], ssem[b], add=True)

    def wait_scatter(c, b):
        pltpu.make_async_copy(rows[b], acc.at[didx.at[c]], ssem[b]).wait()

    for c in range(DEPTH):
        gather(c, c)
    for c in range(DEPTH):
        wait_gather(c, c)
        scatter(c, c)
        gather(c + DEPTH, c + DEPTH)

    @pl.loop(0, (NCHUNK - 2 * DEPTH) // NBUF)
    def _main(p):
        c0 = DEPTH + p * NBUF
        for j in range(NBUF):
            c = c0 + j
            b = (DEPTH + j) % NBUF
            b2 = (b + DEPTH) % NBUF
            wait_gather(c, b)
            scatter(c, b)
            wait_scatter(c + DEPTH - NBUF, b2)
            gather(c + DEPTH, b2)

    for j in range(DEPTH):
        c = NCHUNK - DEPTH + j
        wait_gather(c, c % NBUF)
        scatter(c, c % NBUF)
    for j in range(NBUF):
        c = NCHUNK - NBUF + j
        wait_scatter(c, c % NBUF)


@functools.partial(
    pl.kernel,
    out_type=jax.ShapeDtypeStruct((NC, NPAD), jnp.float32),
    mesh=_MESH,
    scratch_types=[
        pltpu.VMEM((NCHUNK, CHUNK), jnp.int32),
        pltpu.VMEM((CHUNK,), jnp.float32),
        pltpu.VMEM((RPT,), jnp.float32),
        pltpu.VMEM_SHARED((NPAD,), jnp.float32),
    ],
    compiler_params=_SC_PARAMS,
)
def _deg_kernel(dst_hbm, out_hbm, didx, ones_v, z_v, acc):
    cid = lax.axis_index("c")
    sid = lax.axis_index("s")
    wid = sid * NC + cid
    for g in range(CHUNK // 16):
        ones_v[pl.ds(g * 16, 16)] = jnp.ones((16,), jnp.float32)

    @pl.loop(0, RPT // 16)
    def _zero(i):
        z_v[pl.ds(pl.multiple_of(i * 16, 16), 16)] = jnp.zeros((16,), jnp.float32)

    pltpu.sync_copy(z_v, acc.at[pl.ds(sid * RPT, RPT)])
    pltpu.sync_copy(dst_hbm.at[wid], didx)
    plsc.subcore_barrier()

    @pl.loop(0, NCHUNK)
    def _scat(ch):
        pltpu.sync_copy(ones_v, acc.at[didx.at[ch]], add=True)

    plsc.subcore_barrier()
    pltpu.sync_copy(
        acc.at[pl.ds(sid * RPT, RPT)], out_hbm.at[cid, pl.ds(sid * RPT, RPT)]
    )


@functools.partial(
    pl.kernel,
    out_type=jax.ShapeDtypeStruct((NC, NPAD, H), jnp.float32),
    mesh=_MESH,
    scratch_types=(
        [
            pltpu.VMEM((NCHUNK, CHUNK), jnp.int32),
            pltpu.VMEM((NCHUNK, CHUNK), jnp.int32),
            pltpu.VMEM((RPT, H), jnp.float32),
            pltpu.VMEM((RPT, H), jnp.float32),
            pltpu.VMEM((RPT,), jnp.float32),
            pltpu.VMEM((RPT,), jnp.float32),
            pltpu.VMEM_SHARED((NPAD, H), jnp.float32),
            pltpu.VMEM_SHARED((NPAD, H), jnp.float32),
        ]
        + [pltpu.VMEM((CHUNK, H), jnp.float32)] * NBUF
        + [pltpu.SemaphoreType.DMA] * (2 * NBUF)
    ),
    compiler_params=_SC_PARAMS,
)
def _agg1_kernel(src_hbm, dst_hbm, h1_hbm, degp_hbm, out_hbm,
                 sidx, didx, hrows, z_v, d0, d1, acc, tab_sh, *rs):
    rows = rs[:NBUF]
    gsem = rs[NBUF:2 * NBUF]
    ssem = rs[2 * NBUF:]
    cid = lax.axis_index("c")
    sid = lax.axis_index("s")
    wid = sid * NC + cid
    my = pl.ds(sid * RPT, RPT)

    pltpu.sync_copy(h1_hbm.at[my], hrows)
    pltpu.sync_copy(degp_hbm.at[0, my], d0)
    pltpu.sync_copy(degp_hbm.at[1, my], d1)
    pltpu.sync_copy(src_hbm.at[wid], sidx)
    pltpu.sync_copy(dst_hbm.at[wid], didx)

    @pl.loop(0, RPT // 16)
    def _scale(g):
        dis = _dis_group(d0, d1, g)
        r0 = pl.multiple_of(g * 16, 16)
        for i in range(16):
            hrows[r0 + i, :] = hrows[r0 + i, :] * _lane_bcast(dis, i)

    pltpu.sync_copy(hrows, tab_sh.at[my])

    @pl.loop(0, RPT)
    def _zero(i):
        z_v[i, :] = jnp.zeros((16,), jnp.float32)

    pltpu.sync_copy(z_v, acc.at[my])
    plsc.subcore_barrier()

    _edge_ring(tab_sh, acc, sidx, didx, rows, gsem, ssem)

    plsc.subcore_barrier()
    pltpu.sync_copy(acc.at[my], out_hbm.at[cid, my])


@functools.partial(
    pl.kernel,
    out_type=[
        jax.ShapeDtypeStruct((NC, NPAD, H), jnp.float32),
        jax.ShapeDtypeStruct((NPAD, H), jnp.float32),
    ],
    mesh=_MESH,
    scratch_types=(
        [
            pltpu.VMEM((NCHUNK, CHUNK), jnp.int32),
            pltpu.VMEM((NCHUNK, CHUNK), jnp.int32),
            pltpu.VMEM((RPT, H), jnp.float32),
            pltpu.VMEM((RPT, H), jnp.float32),
            pltpu.VMEM((RPT, H), jnp.float32),
            pltpu.VMEM((RPT, H), jnp.float32),
            pltpu.VMEM((RPT,), jnp.float32),
            pltpu.VMEM((RPT,), jnp.float32),
            pltpu.VMEM((H,), jnp.float32),
            pltpu.VMEM_SHARED((NPAD, H), jnp.float32),
            pltpu.VMEM_SHARED((NPAD, H), jnp.float32),
        ]
        + [pltpu.VMEM((CHUNK, H), jnp.float32)] * NBUF
        + [pltpu.SemaphoreType.DMA] * (2 * NBUF)
    ),
    compiler_params=_SC_PARAMS,
)
def _agg2_kernel(src_hbm, dst_hbm, aggp1_hbm, h1_hbm, degp_hbm, b1_hbm,
                 out_hbm, z1_hbm,
                 sidx, didx, hrows, zrows, p0, p1, d0, d1, b1v, acc, tab_sh,
                 *rs):
    rows = rs[:NBUF]
    gsem = rs[NBUF:2 * NBUF]
    ssem = rs[2 * NBUF:]
    cid = lax.axis_index("c")
    sid = lax.axis_index("s")
    wid = sid * NC + cid
    my = pl.ds(sid * RPT, RPT)

    pltpu.sync_copy(h1_hbm.at[my], hrows)
    pltpu.sync_copy(aggp1_hbm.at[0, my], p0)
    pltpu.sync_copy(aggp1_hbm.at[1, my], p1)
    pltpu.sync_copy(degp_hbm.at[0, my], d0)
    pltpu.sync_copy(degp_hbm.at[1, my], d1)
    pltpu.sync_copy(b1_hbm, b1v)
    pltpu.sync_copy(src_hbm.at[wid], sidx)
    pltpu.sync_copy(dst_hbm.at[wid], didx)
    b1vec = b1v[...]

    @pl.loop(0, RPT // 16)
    def _epilogue(g):
        dis = _dis_group(d0, d1, g)
        r0 = pl.multiple_of(g * 16, 16)
        for i in range(16):
            disb = _lane_bcast(dis, i)
            pre = (
                disb * (p0[r0 + i, :] + p1[r0 + i, :])
                + (disb * disb) * hrows[r0 + i, :]
                + b1vec
            )
            z = jnp.maximum(pre, 0.0)
            zrows[r0 + i, :] = z
            hrows[r0 + i, :] = z * disb

    pltpu.sync_copy(hrows, tab_sh.at[my])

    @pl.when(cid == 0)
    def _():
        pltpu.sync_copy(zrows, z1_hbm.at[my])

    @pl.loop(0, RPT)
    def _zero(i):
        zrows[i, :] = jnp.zeros((16,), jnp.float32)

    pltpu.sync_copy(zrows, acc.at[my])
    plsc.subcore_barrier()

    _edge_ring(tab_sh, acc, sidx, didx, rows, gsem, ssem)

    plsc.subcore_barrier()
    pltpu.sync_copy(acc.at[my], out_hbm.at[cid, my])


def _tc_mm_body(x_ref, w1_ref, h_ref):
    h_ref[...] = jnp.dot(
        x_ref[...], w1_ref[...], preferred_element_type=jnp.float32
    )


def _tc_fin_body(aggp_ref, z_ref, degp_ref, w2_ref, b2_ref, out_ref):
    deg = degp_ref[0, :] + degp_ref[1, :] + 1.0
    dis = lax.rsqrt(deg)[:, None]
    pre = dis * (aggp_ref[0] + aggp_ref[1]) + (dis * dis) * z_ref[...]
    logits = (
        jnp.dot(pre, w2_ref[...], preferred_element_type=jnp.float32)
        + b2_ref[...][None, :]
    )
    m = jnp.max(logits, axis=1, keepdims=True)
    lse = jnp.log(jnp.sum(jnp.exp(logits - m), axis=1, keepdims=True)) + m
    out_ref[...] = logits - lse


_tc_mm = pl.pallas_call(
    _tc_mm_body,
    out_shape=jax.ShapeDtypeStruct((NPAD, H), jnp.float32),
)

_tc_fin = pl.pallas_call(
    _tc_fin_body,
    out_shape=jax.ShapeDtypeStruct((NPAD, C), jnp.float32),
)


@jax.jit
def kernel(x, edge_index, W1, b1, W2, b2):
    src = edge_index[0].astype(jnp.int32)
    dst = edge_index[1].astype(jnp.int32)
    pad = NW * EPT - E
    srcp = jnp.concatenate([src, jnp.full((pad,), N, jnp.int32)])
    dstp = jnp.concatenate([dst, jnp.full((pad,), N, jnp.int32)])
    srcp = srcp.reshape(NW, NCHUNK, CHUNK)
    dstp = dstp.reshape(NW, NCHUNK, CHUNK)
    xp = jnp.pad(x, ((0, NPAD - N), (0, 0)))

    degp = _deg_kernel(dstp)
    h1 = _tc_mm(xp, W1)
    aggp1 = _agg1_kernel(srcp, dstp, h1, degp)
    aggp2, z1 = _agg2_kernel(srcp, dstp, aggp1, h1, degp, b1)
    return _tc_fin(aggp2, z1, degp, W2, b2)[:N]


# speedup vs baseline: 74.1245x; 1.0914x over previous
import functools

import jax
import jax.numpy as jnp
from jax import lax
from jax.experimental import pallas as pl
from jax.experimental.pallas import tpu as pltpu
from jax.experimental.pallas import tpu_sc as plsc

N = 10000
NPAD = 10240
H = 16
C = 40
E = 320000
NC, NS = 2, 16
NW = NC * NS
CHUNK = 128
NCHUNK = 80
EPT = NCHUNK * CHUNK
NBUF = 8
DEPTH = 4
RPT = NPAD // NS

_MESH = plsc.VectorSubcoreMesh(
    core_axis_name="c", subcore_axis_name="s", num_cores=NC, num_subcores=NS
)

_SC_PARAMS = pltpu.CompilerParams(
    use_tc_tiling_on_sc=False, needs_layout_passes=False
)

_IN_BOUNDS = lax.GatherScatterMode.PROMISE_IN_BOUNDS


def _rsqrt16(x):
    xh = x * 0.5
    i = plsc.bitcast(x, jnp.int32)
    i = jnp.int32(0x5F3759DF) - (i >> 1)
    y = plsc.bitcast(i, jnp.float32)
    for _ in range(3):
        y = y * (1.5 - xh * y * y)
    return y


def _lane_bcast(v, i):
    return jnp.take_along_axis(
        v, jnp.full((16,), i, jnp.int32), axis=0, mode=_IN_BOUNDS
    )


def _dis_group(d0, d1, g):
    r0 = pl.multiple_of(g * 16, 16)
    deg = d0[pl.ds(r0, 16)] + d1[pl.ds(r0, 16)] + 1.0
    return _rsqrt16(deg)


def _edge_ring(tab_sh, acc, sidx, didx, rows, gsem, ssem):
    def gather(c, b):
        pltpu.async_copy(tab_sh.at[sidx.at[c]], rows[b], gsem[b])

    def wait_gather(c, b):
        pltpu.make_async_copy(tab_sh.at[sidx.at[c]], rows[b], gsem[b]).wait()

    def scatter(c, b):
        pltpu.async_copy(rows[b], acc.at[didx.at[c]], ssem[b], add=True)

    def wait_scatter(c, b):
        pltpu.make_async_copy(rows[b], acc.at[didx.at[c]], ssem[b]).wait()

    for c in range(DEPTH):
        gather(c, c)
    for c in range(DEPTH):
        wait_gather(c, c)
        scatter(c, c)
        gather(c + DEPTH, c + DEPTH)

    @pl.loop(0, (NCHUNK - 2 * DEPTH) // NBUF)
    def _main(p):
        c0 = DEPTH + p * NBUF
        for j in range(NBUF):
            c = c0 + j
            b = (DEPTH + j) % NBUF
            b2 = (b + DEPTH) % NBUF
            wait_gather(c, b)
            scatter(c, b)
            wait_scatter(c + DEPTH - NBUF, b2)
            gather(c + DEPTH, b2)

    for j in range(DEPTH):
        c = NCHUNK - DEPTH + j
        wait_gather(c, c % NBUF)
        scatter(c, c % NBUF)
    for j in range(NBUF):
        c = NCHUNK - NBUF + j
        wait_scatter(c, c % NBUF)


@functools.partial(
    pl.kernel,
    out_type=jax.ShapeDtypeStruct((NC, NPAD), jnp.float32),
    mesh=_MESH,
    scratch_types=[
        pltpu.VMEM((NCHUNK, CHUNK), jnp.int32),
        pltpu.VMEM((CHUNK,), jnp.float32),
        pltpu.VMEM((RPT,), jnp.float32),
        pltpu.VMEM_SHARED((NPAD,), jnp.float32),
    ]
    + [pltpu.SemaphoreType.DMA] * NBUF,
    compiler_params=_SC_PARAMS,
)
def _deg_kernel(dst_hbm, out_hbm, didx, ones_v, z_v, acc, *ssem):
    cid = lax.axis_index("c")
    sid = lax.axis_index("s")
    wid = sid * NC + cid
    for g in range(CHUNK // 16):
        ones_v[pl.ds(g * 16, 16)] = jnp.ones((16,), jnp.float32)

    @pl.loop(0, RPT // 16, unroll=8)
    def _zero(i):
        z_v[pl.ds(pl.multiple_of(i * 16, 16), 16)] = jnp.zeros((16,), jnp.float32)

    pltpu.sync_copy(z_v, acc.at[pl.ds(sid * RPT, RPT)])
    pltpu.sync_copy(dst_hbm.at[wid], didx)
    plsc.subcore_barrier()

    def scat(c, b):
        pltpu.async_copy(ones_v, acc.at[didx.at[c]], ssem[b], add=True)

    def wait_scat(c, b):
        pltpu.make_async_copy(ones_v, acc.at[didx.at[c]], ssem[b]).wait()

    for c in range(NBUF):
        scat(c, c)

    @pl.loop(0, (NCHUNK - NBUF) // NBUF)
    def _scat(p):
        c0 = NBUF + p * NBUF
        for j in range(NBUF):
            wait_scat(c0 + j - NBUF, j)
            scat(c0 + j, j)

    for j in range(NBUF):
        wait_scat(NCHUNK - NBUF + j, j)

    plsc.subcore_barrier()
    pltpu.sync_copy(
        acc.at[pl.ds(sid * RPT, RPT)], out_hbm.at[cid, pl.ds(sid * RPT, RPT)]
    )


@functools.partial(
    pl.kernel,
    out_type=jax.ShapeDtypeStruct((NC, NPAD, H), jnp.float32),
    mesh=_MESH,
    scratch_types=(
        [
            pltpu.VMEM((NCHUNK, CHUNK), jnp.int32),
            pltpu.VMEM((NCHUNK, CHUNK), jnp.int32),
            pltpu.VMEM((RPT, H), jnp.float32),
            pltpu.VMEM((RPT, H), jnp.float32),
            pltpu.VMEM((RPT,), jnp.float32),
            pltpu.VMEM((RPT,), jnp.float32),
            pltpu.VMEM_SHARED((NPAD, H), jnp.float32),
            pltpu.VMEM_SHARED((NPAD, H), jnp.float32),
        ]
        + [pltpu.VMEM((CHUNK, H), jnp.float32)] * NBUF
        + [pltpu.SemaphoreType.DMA] * (2 * NBUF)
    ),
    compiler_params=_SC_PARAMS,
)
def _agg1_kernel(src_hbm, dst_hbm, h1_hbm, degp_hbm, out_hbm,
                 sidx, didx, hrows, z_v, d0, d1, acc, tab_sh, *rs):
    rows = rs[:NBUF]
    gsem = rs[NBUF:2 * NBUF]
    ssem = rs[2 * NBUF:]
    cid = lax.axis_index("c")
    sid = lax.axis_index("s")
    wid = sid * NC + cid
    my = pl.ds(sid * RPT, RPT)

    loads = [
        (h1_hbm.at[my], hrows),
        (degp_hbm.at[0, my], d0),
        (degp_hbm.at[1, my], d1),
        (src_hbm.at[wid], sidx),
        (dst_hbm.at[wid], didx),
    ]
    for k, (s, d) in enumerate(loads):
        pltpu.async_copy(s, d, gsem[k])
    for k in range(3):
        pltpu.make_async_copy(loads[k][0], loads[k][1], gsem[k]).wait()

    @pl.loop(0, RPT // 16)
    def _scale(g):
        dis = _dis_group(d0, d1, g)
        r0 = pl.multiple_of(g * 16, 16)
        for i in range(16):
            hrows[r0 + i, :] = hrows[r0 + i, :] * _lane_bcast(dis, i)

    pltpu.sync_copy(hrows, tab_sh.at[my])

    @pl.loop(0, RPT, unroll=8)
    def _zero(i):
        z_v[i, :] = jnp.zeros((16,), jnp.float32)

    pltpu.sync_copy(z_v, acc.at[my])
    for k in range(3, 5):
        pltpu.make_async_copy(loads[k][0], loads[k][1], gsem[k]).wait()
    plsc.subcore_barrier()

    _edge_ring(tab_sh, acc, sidx, didx, rows, gsem, ssem)

    plsc.subcore_barrier()
    pltpu.sync_copy(acc.at[my], out_hbm.at[cid, my])


@functools.partial(
    pl.kernel,
    out_type=[
        jax.ShapeDtypeStruct((NC, NPAD, H), jnp.float32),
        jax.ShapeDtypeStruct((NPAD, H), jnp.float32),
    ],
    mesh=_MESH,
    scratch_types=(
        [
            pltpu.VMEM((NCHUNK, CHUNK), jnp.int32),
            pltpu.VMEM((NCHUNK, CHUNK), jnp.int32),
            pltpu.VMEM((RPT, H), jnp.float32),
            pltpu.VMEM((RPT, H), jnp.float32),
            pltpu.VMEM((RPT, H), jnp.float32),
            pltpu.VMEM((RPT, H), jnp.float32),
            pltpu.VMEM((RPT,), jnp.float32),
            pltpu.VMEM((RPT,), jnp.float32),
            pltpu.VMEM((H,), jnp.float32),
            pltpu.VMEM_SHARED((NPAD, H), jnp.float32),
            pltpu.VMEM_SHARED((NPAD, H), jnp.float32),
        ]
        + [pltpu.VMEM((CHUNK, H), jnp.float32)] * NBUF
        + [pltpu.SemaphoreType.DMA] * (2 * NBUF)
    ),
    compiler_params=_SC_PARAMS,
)
def _agg2_kernel(src_hbm, dst_hbm, aggp1_hbm, h1_hbm, degp_hbm, b1_hbm,
                 out_hbm, z1_hbm,
                 sidx, didx, hrows, zrows, p0, p1, d0, d1, b1v, acc, tab_sh,
                 *rs):
    rows = rs[:NBUF]
    gsem = rs[NBUF:2 * NBUF]
    ssem = rs[2 * NBUF:]
    cid = lax.axis_index("c")
    sid = lax.axis_index("s")
    wid = sid * NC + cid
    my = pl.ds(sid * RPT, RPT)

    loads = [
        (h1_hbm.at[my], hrows),
        (aggp1_hbm.at[0, my], p0),
        (aggp1_hbm.at[1, my], p1),
        (degp_hbm.at[0, my], d0),
        (degp_hbm.at[1, my], d1),
        (b1_hbm, b1v),
        (src_hbm.at[wid], sidx),
        (dst_hbm.at[wid], didx),
    ]
    for k, (s, d) in enumerate(loads):
        pltpu.async_copy(s, d, gsem[k])
    for k in range(6):
        pltpu.make_async_copy(loads[k][0], loads[k][1], gsem[k]).wait()
    b1vec = b1v[...]

    @pl.loop(0, RPT // 16)
    def _epilogue(g):
        dis = _dis_group(d0, d1, g)
        r0 = pl.multiple_of(g * 16, 16)
        for i in range(16):
            disb = _lane_bcast(dis, i)
            pre = (
                disb * (p0[r0 + i, :] + p1[r0 + i, :])
                + (disb * disb) * hrows[r0 + i, :]
                + b1vec
            )
            z = jnp.maximum(pre, 0.0)
            zrows[r0 + i, :] = z
            hrows[r0 + i, :] = z * disb

    pltpu.sync_copy(hrows, tab_sh.at[my])

    @pl.when(cid == 0)
    def _():
        pltpu.sync_copy(zrows, z1_hbm.at[my])

    @pl.loop(0, RPT, unroll=8)
    def _zero(i):
        zrows[i, :] = jnp.zeros((16,), jnp.float32)

    pltpu.sync_copy(zrows, acc.at[my])
    for k in range(6, 8):
        pltpu.make_async_copy(loads[k][0], loads[k][1], gsem[k]).wait()
    plsc.subcore_barrier()

    _edge_ring(tab_sh, acc, sidx, didx, rows, gsem, ssem)

    plsc.subcore_barrier()
    pltpu.sync_copy(acc.at[my], out_hbm.at[cid, my])


def _tc_mm_body(x_ref, w1_ref, h_ref):
    h_ref[...] = jnp.dot(
        x_ref[...], w1_ref[...], preferred_element_type=jnp.float32
    )


def _tc_fin_body(aggp_ref, z_ref, degp_ref, w2_ref, b2_ref, out_ref):
    deg = degp_ref[0, :] + degp_ref[1, :] + 1.0
    dis = lax.rsqrt(deg)[:, None]
    pre = dis * (aggp_ref[0] + aggp_ref[1]) + (dis * dis) * z_ref[...]
    logits = (
        jnp.dot(pre, w2_ref[...], preferred_element_type=jnp.float32)
        + b2_ref[...][None, :]
    )
    m = jnp.max(logits, axis=1, keepdims=True)
    lse = jnp.log(jnp.sum(jnp.exp(logits - m), axis=1, keepdims=True)) + m
    out_ref[...] = logits - lse


_tc_mm = pl.pallas_call(
    _tc_mm_body,
    out_shape=jax.ShapeDtypeStruct((NPAD, H), jnp.float32),
)

_tc_fin = pl.pallas_call(
    _tc_fin_body,
    out_shape=jax.ShapeDtypeStruct((NPAD, C), jnp.float32),
)


@jax.jit
def kernel(x, edge_index, W1, b1, W2, b2):
    src = edge_index[0].astype(jnp.int32)
    dst = edge_index[1].astype(jnp.int32)
    pad = NW * EPT - E
    srcp = jnp.concatenate([src, jnp.full((pad,), N, jnp.int32)])
    dstp = jnp.concatenate([dst, jnp.full((pad,), N, jnp.int32)])
    srcp = srcp.reshape(NW, NCHUNK, CHUNK)
    dstp = dstp.reshape(NW, NCHUNK, CHUNK)
    xp = jnp.pad(x, ((0, NPAD - N), (0, 0)))

    degp = _deg_kernel(dstp)
    h1 = _tc_mm(xp, W1)
    aggp1 = _agg1_kernel(srcp, dstp, h1, degp)
    aggp2, z1 = _agg2_kernel(srcp, dstp, aggp1, h1, degp, b1)
    return _tc_fin(aggp2, z1, degp, W2, b2)[:N]
